# baseline (speedup 1.0000x reference)
import functools

import jax
import jax.numpy as jnp
from jax import lax
from jax.experimental import pallas as pl
from jax.experimental.pallas import tpu as pltpu
from jax.experimental.pallas import tpu_sc as plsc

N = 10000
E = 320000
H = 128
OUT = 64

NC = 2
NS = 16
NW = NC * NS

NP = 10240
K = 80
ROWS_PER_TILE = NP // NW
EMB_CHUNKS = ROWS_PER_TILE // K
EDGES_PER_TILE = E // NW
KE = 80
AGG_CHUNKS = EDGES_PER_TILE // KE
AGG_PAIRS = (AGG_CHUNKS - 1) // 2
ACC_ROWS = NP // NS
L_SHAPE = (2, 1, H)

_mesh = plsc.VectorSubcoreMesh(core_axis_name="c", subcore_axis_name="s")


@functools.partial(
    pl.kernel,
    out_type=jax.ShapeDtypeStruct((NC, NP, H), jnp.float32),
    mesh=_mesh,
    scratch_types=[
        pltpu.VMEM((EDGES_PER_TILE,), jnp.int32),
        pltpu.VMEM((AGG_CHUNKS, KE), jnp.int32),
        pltpu.VMEM((KE, H), jnp.float32),
        pltpu.VMEM((KE, H), jnp.float32),
        pltpu.VMEM_SHARED((NP, H), jnp.float32),
        pltpu.SemaphoreType.DMA,
        pltpu.SemaphoreType.DMA,
    ],
)
def _agg_kernel(m_hbm, src_hbm, dst_hbm, z_hbm, out_hbm,
                srcv, dstv, rows_a, rows_b, acc, sem_a, sem_b):
    c = lax.axis_index("c")
    s = lax.axis_index("s")
    wid = s * NC + c
    pltpu.sync_copy(z_hbm, rows_a.at[pl.ds(0, 80)])
    for t in range(ACC_ROWS // 80):
        pltpu.async_copy(rows_a.at[pl.ds(0, 80)],
                         acc.at[pl.ds(s * ACC_ROWS + t * 80, 80)], sem_b)
    for t in range(ACC_ROWS // 80):
        pltpu.make_async_copy(rows_a.at[pl.ds(0, 80)],
                              acc.at[pl.ds(s * ACC_ROWS + t * 80, 80)],
                              sem_b).wait()
    pltpu.sync_copy(src_hbm.at[wid], srcv)
    pltpu.sync_copy(dst_hbm.at[wid], dstv)

    def sl(j):
        return srcv.at[pl.ds(j * KE, KE)]

    pltpu.async_copy(m_hbm.at[sl(0)], rows_a, sem_a)
    pltpu.async_copy(m_hbm.at[sl(1)], rows_b, sem_b)
    plsc.subcore_barrier()

    def body(p, carry):
        j = 2 * p
        pltpu.make_async_copy(m_hbm.at[sl(j)], rows_a, sem_a).wait()
        pltpu.sync_copy(rows_a, acc.at[dstv.at[j]], add=True)
        pltpu.async_copy(m_hbm.at[sl(j + 2)], rows_a, sem_a)

        pltpu.make_async_copy(m_hbm.at[sl(j + 1)], rows_b, sem_b).wait()
        pltpu.sync_copy(rows_b, acc.at[dstv.at[j + 1]], add=True)

        @pl.when(p < AGG_PAIRS - 1)
        def _():
            pltpu.async_copy(m_hbm.at[sl(j + 3)], rows_b, sem_b)

        return carry

    lax.fori_loop(0, AGG_PAIRS, body, 0)
    pltpu.make_async_copy(m_hbm.at[sl(AGG_CHUNKS - 1)], rows_a,
                          sem_a).wait()
    pltpu.sync_copy(rows_a, acc.at[dstv.at[AGG_CHUNKS - 1]], add=True)
    plsc.subcore_barrier()
    nwb = ACC_ROWS // 80

    def wb_slot(t):
        buf = (rows_a if t % 2 == 0 else rows_b).at[pl.ds(0, 80)]
        wsem = sem_a if t % 2 == 0 else sem_b
        dst = out_hbm.at[c, pl.ds(s * ACC_ROWS + t * 80, 80)]
        return buf, dst, wsem

    for t in range(nwb):
        buf, dst, wsem = wb_slot(t)
        if t >= 2:
            pbuf, pdst, psem = wb_slot(t - 2)
            pltpu.make_async_copy(pbuf, pdst, psem).wait()
        pltpu.sync_copy(acc.at[pl.ds(s * ACC_ROWS + t * 80, 80)], buf)
        pltpu.async_copy(buf, dst, wsem)
    for t in (nwb - 2, nwb - 1):
        buf, dst, wsem = wb_slot(t)
        pltpu.make_async_copy(buf, dst, wsem).wait()


V = 1001
PRE_BLK = 1024


def _tc_pre_body(f0_ref, f1_ref, key_ref, val_ref, mw_ref, mb_ref,
                 rw_ref, rb_ref, m_ref, r_ref):
    cols = lax.broadcasted_iota(jnp.int32, (PRE_BLK, V), 1)
    oh_k = (cols == f0_ref[...]).astype(jnp.float32)
    oh_v = (cols == f1_ref[...]).astype(jnp.float32)
    x = (jnp.dot(oh_k, key_ref[...], preferred_element_type=jnp.float32)
         + jnp.dot(oh_v, val_ref[...], preferred_element_type=jnp.float32))
    m_ref[...] = jnp.dot(x, mw_ref[...],
                         preferred_element_type=jnp.float32) + mb_ref[...]
    r_ref[...] = jnp.dot(x, rw_ref[...],
                         preferred_element_type=jnp.float32) + rb_ref[...]


def _tc_mid_body(r_ref, agg_ref, aw_ref, ab_ref, mw_ref, mb_ref,
                 rw_ref, rb_ref, m_ref, r2_ref):
    h = jnp.maximum(r_ref[...] + agg_ref[0] + agg_ref[1], 0.0)
    x = jnp.maximum(jnp.dot(h, aw_ref[...],
                            preferred_element_type=jnp.float32) + ab_ref[...],
                    0.0)
    m_ref[...] = jnp.dot(x, mw_ref[...],
                         preferred_element_type=jnp.float32) + mb_ref[...]
    r2_ref[...] = jnp.dot(x, rw_ref[...],
                          preferred_element_type=jnp.float32) + rb_ref[...]


def _tc_out_body(r_ref, agg_ref, aw_ref, ab_ref, cw_ref, o_ref):
    h = jnp.maximum(r_ref[pl.ds(0, N), :] + agg_ref[0, pl.ds(0, N), :]
                    + agg_ref[1, pl.ds(0, N), :], 0.0)
    x = jnp.maximum(jnp.dot(h, aw_ref[...],
                            preferred_element_type=jnp.float32) + ab_ref[...],
                    0.0)
    o_ref[...] = jnp.dot(x, cw_ref[...], preferred_element_type=jnp.float32)


def kernel(feats, edge_index, key_emb, val_emb, act_W, act_b,
           root_W, root_b, msg_W, msg_b, cls_W):
    f32 = jnp.float32
    f0 = jnp.pad(feats[:, 0], (0, NP - N)).astype(jnp.int32).reshape(NP, 1)
    f1 = jnp.pad(feats[:, 1], (0, NP - N)).astype(jnp.int32).reshape(NP, 1)
    src = edge_index[0].astype(jnp.int32).reshape(NW, EDGES_PER_TILE)
    dst = edge_index[1].astype(jnp.int32).reshape(NW, AGG_CHUNKS, KE)
    zrows = jnp.zeros((80, H), f32)

    ab = act_b.reshape(1, H)
    mb = msg_b.reshape(L_SHAPE)
    rb = root_b.reshape(L_SHAPE)

    m1, r1 = pl.pallas_call(
        _tc_pre_body,
        grid=(NP // PRE_BLK,),
        in_specs=[
            pl.BlockSpec((PRE_BLK, 1), lambda i: (i, 0)),
            pl.BlockSpec((PRE_BLK, 1), lambda i: (i, 0)),
            pl.BlockSpec((V, H), lambda i: (0, 0)),
            pl.BlockSpec((V, H), lambda i: (0, 0)),
            pl.BlockSpec((H, H), lambda i: (0, 0)),
            pl.BlockSpec((1, H), lambda i: (0, 0)),
            pl.BlockSpec((H, H), lambda i: (0, 0)),
            pl.BlockSpec((1, H), lambda i: (0, 0)),
        ],
        out_specs=(pl.BlockSpec((PRE_BLK, H), lambda i: (i, 0)),
                   pl.BlockSpec((PRE_BLK, H), lambda i: (i, 0))),
        out_shape=(jax.ShapeDtypeStruct((NP, H), f32),
                   jax.ShapeDtypeStruct((NP, H), f32)),
    )(f0, f1, key_emb, val_emb, msg_W[0], mb[0], root_W[0], rb[0])

    agg1 = _agg_kernel(m1, src, dst, zrows)

    m2, r2 = pl.pallas_call(
        _tc_mid_body,
        out_shape=(jax.ShapeDtypeStruct((NP, H), f32),
                   jax.ShapeDtypeStruct((NP, H), f32)),
    )(r1, agg1, act_W, ab, msg_W[1], mb[1], root_W[1], rb[1])

    agg2 = _agg_kernel(m2, src, dst, zrows)

    out = pl.pallas_call(
        _tc_out_body,
        out_shape=jax.ShapeDtypeStruct((N, OUT), f32),
    )(r2, agg2, act_W, ab, cls_W)

    return out


# speedup vs baseline: 1.0215x; 1.0215x over previous
import functools

import jax
import jax.numpy as jnp
from jax import lax
from jax.experimental import pallas as pl
from jax.experimental.pallas import tpu as pltpu
from jax.experimental.pallas import tpu_sc as plsc

N = 10000
E = 320000
H = 128
OUT = 64

NC = 2
NS = 16
NW = NC * NS

NP = 10240
K = 80
ROWS_PER_TILE = NP // NW
EMB_CHUNKS = ROWS_PER_TILE // K
EDGES_PER_TILE = E // NW
KE = 80
AGG_CHUNKS = EDGES_PER_TILE // KE
AGG_PAIRS = (AGG_CHUNKS - 1) // 2
ACC_ROWS = NP // NS
L_SHAPE = (2, 1, H)

_mesh = plsc.VectorSubcoreMesh(core_axis_name="c", subcore_axis_name="s")


@functools.partial(
    pl.kernel,
    out_type=(
        jax.ShapeDtypeStruct((NP, H), jnp.float32),
        jax.ShapeDtypeStruct((NP, H), jnp.float32),
    ),
    mesh=_mesh,
    scratch_types=[
        pltpu.VMEM((EMB_CHUNKS, K), jnp.int32),
        pltpu.VMEM((EMB_CHUNKS, K), jnp.int32),
        pltpu.VMEM((EMB_CHUNKS, K, H), jnp.float32),
        pltpu.VMEM((EMB_CHUNKS, K, H), jnp.float32),
        pltpu.SemaphoreType.DMA,
        pltpu.SemaphoreType.DMA,
    ],
)
def _emb_kernel(f0_hbm, f1_hbm, key_hbm, val_hbm, xk_hbm, xv_hbm,
                i0v, i1v, rows_k, rows_v, sem_g, sem_w):
    c = lax.axis_index("c")
    s = lax.axis_index("s")
    wid = s * NC + c
    pltpu.sync_copy(f0_hbm.at[wid], i0v)
    pltpu.sync_copy(f1_hbm.at[wid], i1v)
    for j in range(EMB_CHUNKS):
        pltpu.async_copy(key_hbm.at[i0v.at[j]], rows_k.at[j], sem_g)
        pltpu.async_copy(val_hbm.at[i1v.at[j]], rows_v.at[j], sem_g)
    for j in range(EMB_CHUNKS):
        pltpu.make_async_copy(key_hbm.at[i0v.at[j]], rows_k.at[j],
                              sem_g).wait()
        pltpu.make_async_copy(val_hbm.at[i1v.at[j]], rows_v.at[j],
                              sem_g).wait()
    for j in range(EMB_CHUNKS):
        row0 = (wid * EMB_CHUNKS + j) * K
        pltpu.async_copy(rows_k.at[j], xk_hbm.at[pl.ds(row0, K)], sem_w)
        pltpu.async_copy(rows_v.at[j], xv_hbm.at[pl.ds(row0, K)], sem_w)
    for j in range(EMB_CHUNKS):
        row0 = (wid * EMB_CHUNKS + j) * K
        pltpu.make_async_copy(rows_k.at[j], xk_hbm.at[pl.ds(row0, K)],
                              sem_w).wait()
        pltpu.make_async_copy(rows_v.at[j], xv_hbm.at[pl.ds(row0, K)],
                              sem_w).wait()


@functools.partial(
    pl.kernel,
    out_type=jax.ShapeDtypeStruct((NC, NP, H), jnp.float32),
    mesh=_mesh,
    scratch_types=[
        pltpu.VMEM((EDGES_PER_TILE,), jnp.int32),
        pltpu.VMEM((AGG_CHUNKS, KE), jnp.int32),
        pltpu.VMEM((KE, H), jnp.float32),
        pltpu.VMEM((KE, H), jnp.float32),
        pltpu.VMEM_SHARED((NP, H), jnp.float32),
        pltpu.SemaphoreType.DMA,
        pltpu.SemaphoreType.DMA,
    ],
)
def _agg_kernel(m_hbm, src_hbm, dst_hbm, z_hbm, out_hbm,
                srcv, dstv, rows_a, rows_b, acc, sem_a, sem_b):
    c = lax.axis_index("c")
    s = lax.axis_index("s")
    wid = s * NC + c
    pltpu.sync_copy(z_hbm, rows_a.at[pl.ds(0, 80)])
    for t in range(ACC_ROWS // 80):
        pltpu.async_copy(rows_a.at[pl.ds(0, 80)],
                         acc.at[pl.ds(s * ACC_ROWS + t * 80, 80)], sem_b)
    for t in range(ACC_ROWS // 80):
        pltpu.make_async_copy(rows_a.at[pl.ds(0, 80)],
                              acc.at[pl.ds(s * ACC_ROWS + t * 80, 80)],
                              sem_b).wait()
    pltpu.sync_copy(src_hbm.at[wid], srcv)
    pltpu.sync_copy(dst_hbm.at[wid], dstv)

    def sl(j):
        return srcv.at[pl.ds(j * KE, KE)]

    pltpu.async_copy(m_hbm.at[sl(0)], rows_a, sem_a)
    pltpu.async_copy(m_hbm.at[sl(1)], rows_b, sem_b)
    plsc.subcore_barrier()

    def body(p, carry):
        j = 2 * p
        pltpu.make_async_copy(m_hbm.at[sl(j)], rows_a, sem_a).wait()
        pltpu.sync_copy(rows_a, acc.at[dstv.at[j]], add=True)
        pltpu.async_copy(m_hbm.at[sl(j + 2)], rows_a, sem_a)

        pltpu.make_async_copy(m_hbm.at[sl(j + 1)], rows_b, sem_b).wait()
        pltpu.sync_copy(rows_b, acc.at[dstv.at[j + 1]], add=True)

        @pl.when(p < AGG_PAIRS - 1)
        def _():
            pltpu.async_copy(m_hbm.at[sl(j + 3)], rows_b, sem_b)

        return carry

    lax.fori_loop(0, AGG_PAIRS, body, 0)
    pltpu.make_async_copy(m_hbm.at[sl(AGG_CHUNKS - 1)], rows_a,
                          sem_a).wait()
    pltpu.sync_copy(rows_a, acc.at[dstv.at[AGG_CHUNKS - 1]], add=True)
    plsc.subcore_barrier()
    nwb = ACC_ROWS // 80

    def wb_slot(t):
        buf = (rows_a if t % 2 == 0 else rows_b).at[pl.ds(0, 80)]
        wsem = sem_a if t % 2 == 0 else sem_b
        dst = out_hbm.at[c, pl.ds(s * ACC_ROWS + t * 80, 80)]
        return buf, dst, wsem

    for t in range(nwb):
        buf, dst, wsem = wb_slot(t)
        if t >= 2:
            pbuf, pdst, psem = wb_slot(t - 2)
            pltpu.make_async_copy(pbuf, pdst, psem).wait()
        pltpu.sync_copy(acc.at[pl.ds(s * ACC_ROWS + t * 80, 80)], buf)
        pltpu.async_copy(buf, dst, wsem)
    for t in (nwb - 2, nwb - 1):
        buf, dst, wsem = wb_slot(t)
        pltpu.make_async_copy(buf, dst, wsem).wait()


def _tc_pre_body(xk_ref, xv_ref, mw_ref, mb_ref, rw_ref, rb_ref,
                 m_ref, r_ref):
    x = xk_ref[...] + xv_ref[...]
    m_ref[...] = jnp.dot(x, mw_ref[0],
                         preferred_element_type=jnp.float32) + mb_ref[0]
    r_ref[...] = jnp.dot(x, rw_ref[0],
                         preferred_element_type=jnp.float32) + rb_ref[0]


def _wspecs(li):
    return (pl.BlockSpec((1, H, H), lambda i: (li, 0, 0)),
            pl.BlockSpec((1, 1, H), lambda i: (li, 0, 0)))


def _tc_mid_body(r_ref, agg_ref, aw_ref, ab_ref, mw_ref, mb_ref,
                 rw_ref, rb_ref, m_ref, r2_ref):
    h = jnp.maximum(r_ref[...] + agg_ref[0] + agg_ref[1], 0.0)
    x = jnp.maximum(jnp.dot(h, aw_ref[...],
                            preferred_element_type=jnp.float32) + ab_ref[...],
                    0.0)
    m_ref[...] = jnp.dot(x, mw_ref[0],
                         preferred_element_type=jnp.float32) + mb_ref[0]
    r2_ref[...] = jnp.dot(x, rw_ref[0],
                          preferred_element_type=jnp.float32) + rb_ref[0]


def _tc_out_body(r_ref, agg_ref, aw_ref, ab_ref, cw_ref, o_ref):
    h = jnp.maximum(r_ref[pl.ds(0, N), :] + agg_ref[0, pl.ds(0, N), :]
                    + agg_ref[1, pl.ds(0, N), :], 0.0)
    x = jnp.maximum(jnp.dot(h, aw_ref[...],
                            preferred_element_type=jnp.float32) + ab_ref[...],
                    0.0)
    o_ref[...] = jnp.dot(x, cw_ref[...], preferred_element_type=jnp.float32)


def kernel(feats, edge_index, key_emb, val_emb, act_W, act_b,
           root_W, root_b, msg_W, msg_b, cls_W):
    f32 = jnp.float32
    f0 = jnp.pad(feats[:, 0], (0, NP - N)).astype(jnp.int32).reshape(
        NW, EMB_CHUNKS, K)
    f1 = jnp.pad(feats[:, 1], (0, NP - N)).astype(jnp.int32).reshape(
        NW, EMB_CHUNKS, K)
    src = edge_index[0].astype(jnp.int32).reshape(NW, EDGES_PER_TILE)
    dst = edge_index[1].astype(jnp.int32).reshape(NW, AGG_CHUNKS, KE)
    zrows = jnp.zeros((80, H), f32)

    xk, xv = _emb_kernel(f0, f1, key_emb, val_emb)

    ab = act_b.reshape(1, H)
    nph = pl.BlockSpec((NP, H), lambda i: (0, 0))
    w0, b0 = _wspecs(0)
    w1, b1 = _wspecs(1)

    m1, r1 = pl.pallas_call(
        _tc_pre_body,
        grid=(1,),
        in_specs=[nph, nph, w0, b0, w0, b0],
        out_specs=(nph, nph),
        out_shape=(jax.ShapeDtypeStruct((NP, H), f32),
                   jax.ShapeDtypeStruct((NP, H), f32)),
    )(xk, xv, msg_W, msg_b.reshape(L_SHAPE), root_W, root_b.reshape(L_SHAPE))

    agg1 = _agg_kernel(m1, src, dst, zrows)

    m2, r2 = pl.pallas_call(
        _tc_mid_body,
        grid=(1,),
        in_specs=[nph, pl.BlockSpec((NC, NP, H), lambda i: (0, 0, 0)),
                  pl.BlockSpec((H, H), lambda i: (0, 0)),
                  pl.BlockSpec((1, H), lambda i: (0, 0)),
                  w1, b1, w1, b1],
        out_specs=(nph, nph),
        out_shape=(jax.ShapeDtypeStruct((NP, H), f32),
                   jax.ShapeDtypeStruct((NP, H), f32)),
    )(r1, agg1, act_W, ab, msg_W, msg_b.reshape(L_SHAPE),
      root_W, root_b.reshape(L_SHAPE))

    agg2 = _agg_kernel(m2, src, dst, zrows)

    out = pl.pallas_call(
        _tc_out_body,
        out_shape=jax.ShapeDtypeStruct((N, OUT), f32),
    )(r2, agg2, act_W, ab, cls_W)

    return out
